# Initial kernel scaffold; baseline (speedup 1.0000x reference)
import jax
import jax.numpy as jnp
from jax.experimental import pallas as pl


def kernel(proposals):
    raise NotImplementedError("write your pallas kernel here")


# speedup vs baseline: 2.7680x; 2.7680x over previous
import functools

import jax
import jax.numpy as jnp
from jax import lax
from jax.experimental import pallas as pl
from jax.experimental.pallas import tpu as pltpu
from jax.experimental.pallas import tpu_sc as plsc

B = 8
N = 20000
K = 512
NCLS = 5
NITEMS = B * NCLS

R = 4000
NCHUNK = N // R
VPC = R // 16
TOTAL_VREGS = N // 16

ROIS_BUF = 4 * (K + 16)
SCORE_BUF = K + 16


def _sc_call(proposals):
    info = plsc.get_sparse_core_info()
    nc, ns = info.num_cores, info.num_subcores
    nw = nc * ns
    mesh = plsc.VectorSubcoreMesh(core_axis_name="c", subcore_axis_name="s")
    f32 = jnp.float32
    out_type = tuple(
        [jax.ShapeDtypeStruct((B, 4 * K), f32) for _ in range(NCLS)]
        + [jax.ShapeDtypeStruct((B, K), f32) for _ in range(NCLS)]
    )

    @functools.partial(
        pl.kernel,
        out_type=out_type,
        mesh=mesh,
        compiler_params=pltpu.CompilerParams(
            needs_layout_passes=False, use_tc_tiling_on_sc=False),
        scratch_types=[
            pltpu.VMEM((R * 6,), f32),
            pltpu.VMEM((ROIS_BUF,), f32),
            pltpu.VMEM((SCORE_BUF,), f32),
        ],
    )
    def k(prop_hbm, r1, r2, r3, r4, r5, s1, s2, s3, s4, s5,
          chunk_v, rois_v, score_v):
        rois_outs = (r1, r2, r3, r4, r5)
        score_outs = (s1, s2, s3, s4, s5)
        wid = lax.axis_index("s") * nc + lax.axis_index("c")
        iota16 = lax.iota(jnp.int32, 16)
        zero16 = jnp.zeros((16,), f32)

        def process(item):
            b = item // NCLS
            ccls = item % NCLS + 1
            c_f = jnp.broadcast_to(ccls.astype(f32), (16,))

            for j in range(ROIS_BUF // 16):
                rois_v[pl.ds(16 * j, 16)] = zero16
            for j in range(SCORE_BUF // 16):
                score_v[pl.ds(16 * j, 16)] = zero16

            def body(g, ptr):
                fbase = (g * 16 + iota16) * 6
                cls = plsc.load_gather(chunk_v, [fbase])
                m = cls == c_f
                mi = jnp.where(m, 1, 0)
                pos = plsc.cumsum(mi)
                tgt = ptr + pos - 1
                ms = m & (tgt < K)
                sv = plsc.load_gather(chunk_v, [fbase + 1])
                plsc.store_scatter(score_v, [tgt], sv, mask=ms)
                for q in range(4):
                    v = plsc.load_gather(chunk_v, [fbase + 2 + q])
                    plsc.store_scatter(rois_v, [4 * tgt + q], v, mask=ms)
                return ptr + jnp.sum(mi)

            def scan_chunk(ch, ptr):
                pltpu.sync_copy(prop_hbm.at[b, pl.ds(ch * R * 6, R * 6)], chunk_v)
                return lax.fori_loop(0, VPC, body, ptr)

            ptr = jnp.int32(0)
            for ch in range(NCHUNK):
                ptr = lax.cond(ptr < K,
                               lambda p, ch=ch: scan_chunk(ch, p),
                               lambda p: p, ptr)

            for cc in range(NCLS):
                def _write(ro=rois_outs[cc], so=score_outs[cc]):
                    pltpu.sync_copy(rois_v.at[pl.ds(0, 4 * K)], ro.at[b])
                    pltpu.sync_copy(score_v.at[pl.ds(0, K)], so.at[b])
                pl.when(ccls == cc + 1)(_write)

        process(wid)
        if NITEMS > nw:
            process(jnp.where(wid + nw < NITEMS, wid + nw, wid))

    return k(proposals.reshape(B, N * 6))


def kernel(proposals):
    outs = _sc_call(proposals)
    rois = tuple(o.reshape(B, K, 4) for o in outs[:NCLS])
    return rois + tuple(outs[NCLS:])
